# baseline (speedup 1.0000x reference)
import functools

import jax
import jax.numpy as jnp
from jax import lax
from jax.experimental import pallas as pl
from jax.experimental.pallas import tpu as pltpu
from jax.experimental.pallas import tpu_sc as plsc

N = 10000
E = 160000
HID = 128
EPS = 1e-5

BN = 1000
BE = 1000

_EC = E // 2
_CHUNK = 128
_CHUNKG = 64
_NCHUNK = _EC // _CHUNK
_NCHUNKG = _EC // _CHUNKG
_RPT = 624
_RLAST = N - 15 * _RPT

_f32 = jnp.float32


def _ln_rows(t, g, b):
    mu = jnp.mean(t, axis=-1, keepdims=True)
    var = jnp.mean((t - mu) ** 2, axis=-1, keepdims=True)
    return (t - mu) / jnp.sqrt(var + EPS) * g + b


def _layer_prep(h, wg_ref, as_a_ref, ad_a_ref, xh_ref, as_ref, ad_ref, m_ref, pid):
    xh = jnp.dot(h, wg_ref[...], preferred_element_type=_f32)
    xh_ref[...] = xh
    a_s = jnp.dot(xh, as_a_ref[...], preferred_element_type=_f32)
    a_d = jnp.dot(xh, ad_a_ref[...], preferred_element_type=_f32)
    z = jnp.zeros_like(a_s)
    as_ref[...] = jnp.concatenate([a_s, z], axis=1)
    ad_ref[...] = jnp.concatenate([a_d, z], axis=1)
    bm = jnp.max(a_s, axis=0, keepdims=True)
    bm16 = jnp.concatenate([bm, jnp.zeros_like(bm)], axis=1)

    @pl.when(pid == 0)
    def _():
        m_ref[...] = bm16

    @pl.when(pid != 0)
    def _():
        m_ref[...] = jnp.maximum(m_ref[...], bm16)


def _pre0_body(x_ref, wenc_ref, benc_ref, lng_ref, lnb_ref,
               wg_ref, as_a_ref, ad_a_ref,
               h_ref, xh_ref, as_ref, ad_ref, m_ref):
    pid = pl.program_id(0)
    t = jnp.dot(x_ref[...], wenc_ref[...], preferred_element_type=_f32) + benc_ref[...]
    h = jnp.maximum(_ln_rows(t, lng_ref[...], lnb_ref[...]), 0.0)
    h_ref[...] = h
    _layer_prep(h, wg_ref, as_a_ref, ad_a_ref, xh_ref, as_ref, ad_ref, m_ref, pid)


def _pre_next_body(use_res, y_ref, st_ref, hp_ref, gng_ref, gnb_ref,
                   wres_ref, bres_ref, wg_ref, as_a_ref, ad_a_ref,
                   h_ref, xh_ref, as_ref, ad_ref, m_ref):
    pid = pl.program_id(0)
    s1 = st_ref[0:1, 0:1]
    s2 = st_ref[0:1, 1:2]
    mean = s1 / (N * HID)
    var = jnp.maximum(s2 / (N * HID) - mean * mean, 0.0)
    g = (y_ref[...] - mean) / (jnp.sqrt(var) + EPS) * gng_ref[...] + gnb_ref[...]
    if use_res:
        g = g + jnp.dot(hp_ref[...], wres_ref[...], preferred_element_type=_f32) + bres_ref[...]
    h = jnp.maximum(g, 0.0)
    h_ref[...] = h
    _layer_prep(h, wg_ref, as_a_ref, ad_a_ref, xh_ref, as_ref, ad_ref, m_ref, pid)


def _post_body(y_ref, st_ref, hp_ref, gng_ref, gnb_ref, wres_ref, bres_ref, h_ref):
    s1 = st_ref[0:1, 0:1]
    s2 = st_ref[0:1, 1:2]
    mean = s1 / (N * HID)
    var = jnp.maximum(s2 / (N * HID) - mean * mean, 0.0)
    g = (y_ref[...] - mean) / (jnp.sqrt(var) + EPS) * gng_ref[...] + gnb_ref[...]
    g = g + jnp.dot(hp_ref[...], wres_ref[...], preferred_element_type=_f32) + bres_ref[...]
    h_ref[...] = jnp.maximum(g, 0.0)


def _comb_body(acc_ref, den_ref, xh_ref, as_ref, ad_ref, m_ref, bias_ref, t_ref,
               y_ref, st_ref):
    pid = pl.program_id(0)
    acc = acc_ref[0] + acc_ref[1]
    den = den_ref[0, :, 0:8] + den_ref[1, :, 0:8]
    a_s = as_ref[:, 0:8]
    a_d = ad_ref[:, 0:8]
    t = a_s + a_d
    lr = jnp.maximum(t, 0.2 * t)
    u = a_d + m_ref[0:1, 0:8]
    shift = jnp.maximum(u, 0.2 * u)
    wself = jnp.exp(lr - shift)
    dinv = 1.0 / (den + wself + 1e-16)
    bmat = t_ref[...]
    y = (acc + xh_ref[...] * jnp.dot(wself, bmat, preferred_element_type=_f32)) \
        * jnp.dot(dinv, bmat, preferred_element_type=_f32) + bias_ref[...]
    y_ref[...] = y
    s1 = jnp.sum(y)
    s2 = jnp.sum(y * y)
    lanes = lax.broadcasted_iota(jnp.int32, (1, HID), 1)
    vec = jnp.where(lanes == 0, s1, jnp.where(lanes == 1, s2, 0.0))

    @pl.when(pid == 0)
    def _():
        st_ref[...] = vec

    @pl.when(pid != 0)
    def _():
        st_ref[...] = st_ref[...] + vec


def _bdot(a, b):
    return jnp.dot(a.astype(jnp.bfloat16), b.astype(jnp.bfloat16),
                   preferred_element_type=_f32)


def _cls_body(sf_ref, df_ref, ea_ref,
              we_ref, be_ref, elng_ref, elnb_ref,
              wgs_ref, wgd_ref, wge_ref, bg_ref,
              w1s_ref, w1d_ref, b1_ref, l1g_ref, l1b_ref,
              w2_ref, b2_ref, l2g_ref, l2b_ref,
              w3_ref, b3_ref,
              o_ref):
    sf = sf_ref[...]
    df = df_ref[...]
    ef = jnp.dot(ea_ref[...], we_ref[...], preferred_element_type=_f32) + be_ref[...]
    ef = jnp.maximum(_ln_rows(ef, elng_ref[...], elnb_ref[...]), 0.0)
    gate = _bdot(sf, wgs_ref[...]) + _bdot(df, wgd_ref[...]) \
        + _bdot(ef, wge_ref[...]) + bg_ref[...]
    gate = jax.nn.sigmoid(gate)
    gef = gate * ef
    s2 = sf + gef
    d2 = df + gef
    z1 = _bdot(s2, w1s_ref[...]) + _bdot(d2, w1d_ref[...]) + b1_ref[...]
    z1 = jnp.maximum(_ln_rows(z1, l1g_ref[...], l1b_ref[...]), 0.0)
    z2 = _bdot(z1, w2_ref[...]) + b2_ref[...]
    z2 = jnp.maximum(_ln_rows(z2, l2g_ref[...], l2b_ref[...]), 0.0)
    o = jnp.dot(z2, w3_ref[...], preferred_element_type=_f32) + b3_ref[...]
    o_ref[...] = o[:, 0:2]


_TC_PARAMS = pltpu.CompilerParams(dimension_semantics=("arbitrary",))


def _row_spec(bn, width):
    return pl.BlockSpec((bn, width), lambda i: (i, 0))


def _full_spec(shape):
    nd = len(shape)
    return pl.BlockSpec(shape, lambda i: (0,) * nd)


def _run_pre0(x, wenc, benc, lng, lnb, wg, as_a, ad_a):
    grid = (N // BN,)
    return pl.pallas_call(
        _pre0_body,
        grid=grid,
        in_specs=[_row_spec(BN, HID)] + [_full_spec(a.shape) for a in
                                         (wenc, benc, lng, lnb, wg, as_a, ad_a)],
        out_specs=[_row_spec(BN, HID), _row_spec(BN, HID),
                   _row_spec(BN, 16), _row_spec(BN, 16), _full_spec((1, 16))],
        out_shape=[jax.ShapeDtypeStruct((N, HID), _f32),
                   jax.ShapeDtypeStruct((N, HID), _f32),
                   jax.ShapeDtypeStruct((N, 16), _f32),
                   jax.ShapeDtypeStruct((N, 16), _f32),
                   jax.ShapeDtypeStruct((1, 16), _f32)],
        compiler_params=_TC_PARAMS,
    )(x, wenc, benc, lng, lnb, wg, as_a, ad_a)


def _run_pre_next(use_res, y, st, hp, gng, gnb, wres, bres, wg, as_a, ad_a):
    grid = (N // BN,)
    return pl.pallas_call(
        functools.partial(_pre_next_body, use_res),
        grid=grid,
        in_specs=[_row_spec(BN, HID), _full_spec((1, HID)), _row_spec(BN, HID)]
        + [_full_spec(a.shape) for a in (gng, gnb, wres, bres, wg, as_a, ad_a)],
        out_specs=[_row_spec(BN, HID), _row_spec(BN, HID),
                   _row_spec(BN, 16), _row_spec(BN, 16), _full_spec((1, 16))],
        out_shape=[jax.ShapeDtypeStruct((N, HID), _f32),
                   jax.ShapeDtypeStruct((N, HID), _f32),
                   jax.ShapeDtypeStruct((N, 16), _f32),
                   jax.ShapeDtypeStruct((N, 16), _f32),
                   jax.ShapeDtypeStruct((1, 16), _f32)],
        compiler_params=_TC_PARAMS,
    )(y, st, hp, gng, gnb, wres, bres, wg, as_a, ad_a)


def _run_post(y, st, hp, gng, gnb, wres, bres):
    grid = (N // BN,)
    return pl.pallas_call(
        _post_body,
        grid=grid,
        in_specs=[_row_spec(BN, HID), _full_spec((1, HID)), _row_spec(BN, HID)]
        + [_full_spec(a.shape) for a in (gng, gnb, wres, bres)],
        out_specs=_row_spec(BN, HID),
        out_shape=jax.ShapeDtypeStruct((N, HID), _f32),
        compiler_params=_TC_PARAMS,
    )(y, st, hp, gng, gnb, wres, bres)


def _run_comb(acc, den, xh, as16, ad16, m, bias, bmat):
    grid = (N // BN,)
    return pl.pallas_call(
        _comb_body,
        grid=grid,
        in_specs=[pl.BlockSpec((2, BN, HID), lambda i: (0, i, 0)),
                  pl.BlockSpec((2, BN, 16), lambda i: (0, i, 0)),
                  _row_spec(BN, HID), _row_spec(BN, 16), _row_spec(BN, 16),
                  _full_spec((1, 16)), _full_spec((1, HID)), _full_spec((8, HID))],
        out_specs=[_row_spec(BN, HID), _full_spec((1, HID))],
        out_shape=[jax.ShapeDtypeStruct((N, HID), _f32),
                   jax.ShapeDtypeStruct((1, HID), _f32)],
        compiler_params=_TC_PARAMS,
    )(acc, den, xh, as16, ad16, m, bias, bmat)


def _run_cls(sf, df, ea, weights):
    grid = (E // BE,)
    return pl.pallas_call(
        _cls_body,
        grid=grid,
        in_specs=[_row_spec(BE, HID), _row_spec(BE, HID), _row_spec(BE, 16)]
        + [_full_spec(w.shape) for w in weights],
        out_specs=_row_spec(BE, 2),
        out_shape=jax.ShapeDtypeStruct((E, 2), _f32),
        compiler_params=_TC_PARAMS,
    )(sf, df, ea, *weights)


def _sc_gat_body(n_heads, ei_h, as_h, ad_h, xh_h, m_h, z128_h, z16_h,
                 acc_o, den_o,
                 sidx, didx, asg, adg, wv, xg, mv, acc_sp, den_sp, gsem, ssem):
    c = lax.axis_index("c")
    s = lax.axis_index("s")
    r0 = s * _RPT

    @pl.when(s < 15)
    def _():
        pltpu.sync_copy(z128_h.at[pl.ds(0, _RPT)], acc_sp.at[pl.ds(r0, _RPT)])
        pltpu.sync_copy(z16_h.at[pl.ds(0, _RPT)], den_sp.at[pl.ds(r0, _RPT)])

    @pl.when(s == 15)
    def _():
        pltpu.sync_copy(z128_h.at[pl.ds(0, _RLAST)], acc_sp.at[pl.ds(r0, _RLAST)])
        pltpu.sync_copy(z16_h.at[pl.ds(0, _RLAST)], den_sp.at[pl.ds(r0, _RLAST)])

    pltpu.sync_copy(m_h, mv)
    plsc.subcore_barrier()
    mvec = mv[0]
    nch = jnp.where(s < 2, 79, 78)

    def issue(j, b):
        base = c * _EC + (16 * j + s) * _CHUNKG
        pltpu.sync_copy(ei_h.at[0, pl.ds(base, _CHUNKG)], sidx[b])
        pltpu.sync_copy(ei_h.at[1, pl.ds(base, _CHUNKG)], didx[b])
        pltpu.async_copy(as_h.at[sidx[b]], asg[b], gsem[b])
        pltpu.async_copy(ad_h.at[didx[b]], adg[b], gsem[b])
        pltpu.async_copy(xh_h.at[sidx[b]], xg[b], gsem[b])

    def wait_gathers(b):
        pltpu.make_async_copy(as_h.at[sidx[b]], asg[b], gsem[b]).wait()
        pltpu.make_async_copy(ad_h.at[didx[b]], adg[b], gsem[b]).wait()
        pltpu.make_async_copy(xh_h.at[sidx[b]], xg[b], gsem[b]).wait()

    def wait_scatters(b):
        pltpu.make_async_copy(wv[b], den_sp.at[didx[b]], ssem[b]).wait()
        pltpu.make_async_copy(xg[b], acc_sp.at[didx[b]], ssem[b]).wait()

    def compute_and_scatter(b):
        @plsc.parallel_loop(0, _CHUNKG, 1, unroll=4)
        def _(r):
            a = asg[b][r]
            d = adg[b][r]
            t = a + d
            u = d + mvec
            w = jnp.exp(jnp.maximum(t, 0.2 * t) - jnp.maximum(u, 0.2 * u))
            wv[b][r] = w
            for k in range(8):
                kk = k if n_heads == 8 else 0
                xg[b][r, pl.ds(16 * k, 16)] = xg[b][r, pl.ds(16 * k, 16)] * w[kk]

        pltpu.async_copy(wv[b], den_sp.at[didx[b]], ssem[b], add=True)
        pltpu.async_copy(xg[b], acc_sp.at[didx[b]], ssem[b], add=True)

    issue(0, 0)

    def pair(jj, carry):
        j0 = 2 * jj
        j1 = j0 + 1
        j2 = j0 + 2

        @pl.when(j1 < nch)
        def _():
            @pl.when(j1 >= 3)
            def _():
                wait_scatters(1)
            issue(j1, 1)

        @pl.when(j0 < nch)
        def _():
            wait_gathers(0)
            compute_and_scatter(0)

        @pl.when(j2 < nch)
        def _():
            wait_scatters(0)
            issue(j2, 0)

        @pl.when(j1 < nch)
        def _():
            wait_gathers(1)
            compute_and_scatter(1)

        return carry

    lax.fori_loop(0, 40, pair, 0)
    wait_scatters(0)
    wait_scatters(1)
    plsc.subcore_barrier()

    @pl.when(s < 15)
    def _():
        pltpu.sync_copy(acc_sp.at[pl.ds(r0, _RPT)], acc_o.at[c, pl.ds(r0, _RPT)])
        pltpu.sync_copy(den_sp.at[pl.ds(r0, _RPT)], den_o.at[c, pl.ds(r0, _RPT)])

    @pl.when(s == 15)
    def _():
        pltpu.sync_copy(acc_sp.at[pl.ds(r0, _RLAST)], acc_o.at[c, pl.ds(r0, _RLAST)])
        pltpu.sync_copy(den_sp.at[pl.ds(r0, _RLAST)], den_o.at[c, pl.ds(r0, _RLAST)])


def _run_sc_gat(n_heads, edge_index, as16, ad16, xh, m16, z128, z16):
    mesh = plsc.VectorSubcoreMesh(core_axis_name="c", subcore_axis_name="s")
    fn = functools.partial(
        pl.kernel,
        out_type=[jax.ShapeDtypeStruct((2, N, HID), _f32),
                  jax.ShapeDtypeStruct((2, N, 16), _f32)],
        mesh=mesh,
        scratch_types=[
            [pltpu.VMEM((_CHUNKG,), jnp.int32)] * 2,
            [pltpu.VMEM((_CHUNKG,), jnp.int32)] * 2,
            [pltpu.VMEM((_CHUNKG, 16), _f32)] * 2,
            [pltpu.VMEM((_CHUNKG, 16), _f32)] * 2,
            [pltpu.VMEM((_CHUNKG, 16), _f32)] * 2,
            [pltpu.VMEM((_CHUNKG, HID), _f32)] * 2,
            pltpu.VMEM((1, 16), _f32),
            pltpu.VMEM_SHARED((N, HID), _f32),
            pltpu.VMEM_SHARED((N, 16), _f32),
            [pltpu.SemaphoreType.DMA] * 2,
            [pltpu.SemaphoreType.DMA] * 2,
        ],
        compiler_params=pltpu.CompilerParams(use_tc_tiling_on_sc=False),
    )(functools.partial(_sc_gat_body, n_heads))
    return fn(edge_index, as16, ad16, xh, m16, z128, z16)


def _sc_gather_body(ei_h, h_h, sf_o, df_o, sidx, didx, sbuf, dbuf,
                    gsem, wsem):
    c = lax.axis_index("c")
    s = lax.axis_index("s")
    nch = jnp.where(s == 0, 40, 39)

    def base_of(j):
        return c * _EC + (16 * j + s) * _CHUNK

    def issue(j, b):
        base = base_of(j)
        pltpu.sync_copy(ei_h.at[0, pl.ds(base, _CHUNK)], sidx[b])
        pltpu.sync_copy(ei_h.at[1, pl.ds(base, _CHUNK)], didx[b])
        pltpu.async_copy(h_h.at[sidx[b]], sbuf[b], gsem[b])
        pltpu.async_copy(h_h.at[didx[b]], dbuf[b], gsem[b])

    def wait_gathers(b):
        pltpu.make_async_copy(h_h.at[sidx[b]], sbuf[b], gsem[b]).wait()
        pltpu.make_async_copy(h_h.at[didx[b]], dbuf[b], gsem[b]).wait()

    def writeback(j, b):
        base = base_of(j)
        pltpu.async_copy(sbuf[b], sf_o.at[pl.ds(base, _CHUNK)], wsem[b])
        pltpu.async_copy(dbuf[b], df_o.at[pl.ds(base, _CHUNK)], wsem[b])

    def wait_writeback(j, b):
        base = base_of(j)
        pltpu.make_async_copy(sbuf[b], sf_o.at[pl.ds(base, _CHUNK)], wsem[b]).wait()
        pltpu.make_async_copy(dbuf[b], df_o.at[pl.ds(base, _CHUNK)], wsem[b]).wait()

    issue(0, 0)

    def pair(jj, carry):
        j0 = 2 * jj
        j1 = j0 + 1
        j2 = j0 + 2

        @pl.when(j1 < nch)
        def _():
            @pl.when(j1 >= 3)
            def _():
                wait_writeback(j1, 1)
            issue(j1, 1)

        wait_gathers(0)
        writeback(j0, 0)

        @pl.when(j2 < nch)
        def _():
            wait_writeback(j2, 0)
            issue(j2, 0)

        @pl.when(j1 < nch)
        def _():
            wait_gathers(1)
            writeback(j1, 1)

        return carry

    lax.fori_loop(0, 20, pair, 0)
    wait_writeback(0, 0)
    wait_writeback(0, 1)


def _run_sc_gather(edge_index, h):
    mesh = plsc.VectorSubcoreMesh(core_axis_name="c", subcore_axis_name="s")
    fn = functools.partial(
        pl.kernel,
        out_type=[jax.ShapeDtypeStruct((E, HID), _f32),
                  jax.ShapeDtypeStruct((E, HID), _f32)],
        mesh=mesh,
        scratch_types=[
            [pltpu.VMEM((_CHUNK,), jnp.int32)] * 2,
            [pltpu.VMEM((_CHUNK,), jnp.int32)] * 2,
            [pltpu.VMEM((_CHUNK, HID), _f32)] * 2,
            [pltpu.VMEM((_CHUNK, HID), _f32)] * 2,
            [pltpu.SemaphoreType.DMA] * 2,
            [pltpu.SemaphoreType.DMA] * 2,
        ],
    )(_sc_gather_body)
    return fn(edge_index, h)


def _att_matrix(att):
    heads, ch = att.shape
    eye = jnp.eye(heads, dtype=_f32)
    r = jnp.repeat(eye, ch, axis=0)
    a = att.reshape(-1, 1) * r
    if heads < 8:
        a = jnp.pad(a, ((0, 0), (0, 8 - heads)))
    return a


def kernel(x, edge_index, edge_attr, params):
    p = params

    gat = p['gat']
    heads_cfg = [8, 8, 1]
    as_a = [_att_matrix(gat[i]['att_src']) for i in range(3)]
    ad_a = [_att_matrix(gat[i]['att_dst']) for i in range(3)]
    t8 = jnp.repeat(jnp.eye(8, dtype=_f32), 16, axis=1)
    t1 = jnp.zeros((8, HID), _f32).at[0, :].set(1.0)
    bmats = [t8, t8, t1]

    z128 = jnp.zeros((_RLAST, HID), _f32)
    z16 = jnp.zeros((_RLAST, 16), _f32)

    def row(v):
        return v.reshape(1, -1)

    ne = p['node_enc']
    h0, xh0, as0, ad0, m0 = _run_pre0(
        x, ne['lin']['w'], row(ne['lin']['b']), row(ne['ln_g']), row(ne['ln_b']),
        gat[0]['w'], as_a[0], ad_a[0])

    hs = [h0]
    xh, as16, ad16, m = xh0, as0, ad0, m0
    y = st = None
    for i in range(3):
        acc, den = _run_sc_gat(heads_cfg[i], edge_index, as16, ad16, xh, m,
                               z128, z16)
        y, st = _run_comb(acc, den, xh, as16, ad16, m,
                          row(gat[i]['bias']), bmats[i])
        nm = p['norms'][i]
        if i < 2:
            j = i + 1
            use_res = j > 1
            wres = p['res'][j - 2]['w'] if use_res else jnp.zeros((HID, HID), _f32)
            bres = row(p['res'][j - 2]['b']) if use_res else jnp.zeros((1, HID), _f32)
            hj, xh, as16, ad16, m = _run_pre_next(
                use_res, y, st, hs[-1], row(nm['g']), row(nm['b']),
                wres, bres, gat[j]['w'], as_a[j], ad_a[j])
            hs.append(hj)
        else:
            hf = _run_post(y, st, hs[-1], row(nm['g']), row(nm['b']),
                           p['res'][1]['w'], row(p['res'][1]['b']))

    sf, df = _run_sc_gather(edge_index, hf)

    c = p['cls']
    eg_w = p['edge_gate']['w']
    l1_w = c['lin1']['w']
    w3 = jnp.pad(c['lin3']['w'], ((0, 0), (0, HID - 2)))
    b3 = jnp.pad(c['lin3']['b'], (0, HID - 2))
    ee = p['edge_enc']
    weights = [
        ee['lin']['w'], row(ee['lin']['b']), row(ee['ln_g']), row(ee['ln_b']),
        eg_w[:HID], eg_w[HID:2 * HID], eg_w[2 * HID:], row(p['edge_gate']['b']),
        l1_w[:HID], l1_w[HID:], row(c['lin1']['b']), row(c['ln1_g']), row(c['ln1_b']),
        c['lin2']['w'], row(c['lin2']['b']), row(c['ln2_g']), row(c['ln2_b']),
        w3, b3.reshape(1, -1),
    ]
    return _run_cls(sf, df, edge_attr, weights)


# speedup vs baseline: 1.0228x; 1.0228x over previous
import functools

import jax
import jax.numpy as jnp
from jax import lax
from jax.experimental import pallas as pl
from jax.experimental.pallas import tpu as pltpu
from jax.experimental.pallas import tpu_sc as plsc

N = 10000
E = 160000
HID = 128
EPS = 1e-5

BN = 1000
BE = 640

_EC = E // 2
_CHUNK = 128
_CHUNKG = 64
_NCHUNK = _EC // _CHUNK
_NCHUNKG = _EC // _CHUNKG
_RPT = 624
_RLAST = N - 15 * _RPT

_f32 = jnp.float32


def _ln_rows(t, g, b):
    mu = jnp.mean(t, axis=-1, keepdims=True)
    var = jnp.mean((t - mu) ** 2, axis=-1, keepdims=True)
    return (t - mu) / jnp.sqrt(var + EPS) * g + b


def _layer_prep(h, wg_ref, as_a_ref, ad_a_ref, xh_ref, as_ref, ad_ref, m_ref, pid):
    xh = jnp.dot(h, wg_ref[...], preferred_element_type=_f32)
    xh_ref[...] = xh
    a_s = jnp.dot(xh, as_a_ref[...], preferred_element_type=_f32)
    a_d = jnp.dot(xh, ad_a_ref[...], preferred_element_type=_f32)
    z = jnp.zeros_like(a_s)
    as_ref[...] = jnp.concatenate([a_s, z], axis=1)
    ad_ref[...] = jnp.concatenate([a_d, z], axis=1)
    bm = jnp.max(a_s, axis=0, keepdims=True)
    bm16 = jnp.concatenate([bm, jnp.zeros_like(bm)], axis=1)

    @pl.when(pid == 0)
    def _():
        m_ref[...] = bm16

    @pl.when(pid != 0)
    def _():
        m_ref[...] = jnp.maximum(m_ref[...], bm16)


def _pre0_body(x_ref, wenc_ref, benc_ref, lng_ref, lnb_ref,
               wg_ref, as_a_ref, ad_a_ref,
               h_ref, xh_ref, as_ref, ad_ref, m_ref):
    pid = pl.program_id(0)
    t = jnp.dot(x_ref[...], wenc_ref[...], preferred_element_type=_f32) + benc_ref[...]
    h = jnp.maximum(_ln_rows(t, lng_ref[...], lnb_ref[...]), 0.0)
    h_ref[...] = h
    _layer_prep(h, wg_ref, as_a_ref, ad_a_ref, xh_ref, as_ref, ad_ref, m_ref, pid)


def _pre_next_body(use_res, y_ref, st_ref, hp_ref, gng_ref, gnb_ref,
                   wres_ref, bres_ref, wg_ref, as_a_ref, ad_a_ref,
                   h_ref, xh_ref, as_ref, ad_ref, m_ref):
    pid = pl.program_id(0)
    s1 = st_ref[0:1, 0:1]
    s2 = st_ref[0:1, 1:2]
    mean = s1 / (N * HID)
    var = jnp.maximum(s2 / (N * HID) - mean * mean, 0.0)
    g = (y_ref[...] - mean) / (jnp.sqrt(var) + EPS) * gng_ref[...] + gnb_ref[...]
    if use_res:
        g = g + jnp.dot(hp_ref[...], wres_ref[...], preferred_element_type=_f32) + bres_ref[...]
    h = jnp.maximum(g, 0.0)
    h_ref[...] = h
    _layer_prep(h, wg_ref, as_a_ref, ad_a_ref, xh_ref, as_ref, ad_ref, m_ref, pid)


def _post_body(y_ref, st_ref, hp_ref, gng_ref, gnb_ref, wres_ref, bres_ref, h_ref):
    s1 = st_ref[0:1, 0:1]
    s2 = st_ref[0:1, 1:2]
    mean = s1 / (N * HID)
    var = jnp.maximum(s2 / (N * HID) - mean * mean, 0.0)
    g = (y_ref[...] - mean) / (jnp.sqrt(var) + EPS) * gng_ref[...] + gnb_ref[...]
    g = g + jnp.dot(hp_ref[...], wres_ref[...], preferred_element_type=_f32) + bres_ref[...]
    h_ref[...] = jnp.maximum(g, 0.0)


def _comb_body(acc_ref, den_ref, xh_ref, as_ref, ad_ref, m_ref, bias_ref, t_ref,
               y_ref, st_ref):
    pid = pl.program_id(0)
    acc = acc_ref[0] + acc_ref[1]
    den = den_ref[0, :, 0:8] + den_ref[1, :, 0:8]
    a_s = as_ref[:, 0:8]
    a_d = ad_ref[:, 0:8]
    t = a_s + a_d
    lr = jnp.maximum(t, 0.2 * t)
    u = a_d + m_ref[0:1, 0:8]
    shift = jnp.maximum(u, 0.2 * u)
    wself = jnp.exp(lr - shift)
    dinv = 1.0 / (den + wself + 1e-16)
    bmat = t_ref[...]
    y = (acc + xh_ref[...] * jnp.dot(wself, bmat, preferred_element_type=_f32)) \
        * jnp.dot(dinv, bmat, preferred_element_type=_f32) + bias_ref[...]
    y_ref[...] = y
    s1 = jnp.sum(y)
    s2 = jnp.sum(y * y)
    lanes = lax.broadcasted_iota(jnp.int32, (1, HID), 1)
    vec = jnp.where(lanes == 0, s1, jnp.where(lanes == 1, s2, 0.0))

    @pl.when(pid == 0)
    def _():
        st_ref[...] = vec

    @pl.when(pid != 0)
    def _():
        st_ref[...] = st_ref[...] + vec


def _cls_body(sf_ref, df_ref, ea_ref,
              we_ref, be_ref, elng_ref, elnb_ref,
              wgs_ref, wgd_ref, wge_ref, bg_ref,
              w1s_ref, w1d_ref, b1_ref, l1g_ref, l1b_ref,
              w2_ref, b2_ref, l2g_ref, l2b_ref,
              w3_ref, b3_ref,
              o_ref):
    sf = sf_ref[...]
    df = df_ref[...]
    ef = jnp.dot(ea_ref[...], we_ref[...], preferred_element_type=_f32) + be_ref[...]
    ef = jnp.maximum(_ln_rows(ef, elng_ref[...], elnb_ref[...]), 0.0)
    gate = jnp.dot(sf, wgs_ref[...], preferred_element_type=_f32) \
        + jnp.dot(df, wgd_ref[...], preferred_element_type=_f32) \
        + jnp.dot(ef, wge_ref[...], preferred_element_type=_f32) + bg_ref[...]
    gate = jax.nn.sigmoid(gate)
    gef = gate * ef
    s2 = sf + gef
    d2 = df + gef
    z1 = jnp.dot(s2, w1s_ref[...], preferred_element_type=_f32) \
        + jnp.dot(d2, w1d_ref[...], preferred_element_type=_f32) + b1_ref[...]
    z1 = jnp.maximum(_ln_rows(z1, l1g_ref[...], l1b_ref[...]), 0.0)
    z2 = jnp.dot(z1, w2_ref[...], preferred_element_type=_f32) + b2_ref[...]
    z2 = jnp.maximum(_ln_rows(z2, l2g_ref[...], l2b_ref[...]), 0.0)
    o2 = lax.dot_general(w3_ref[...], z2, (((1,), (1,)), ((), ())),
                         preferred_element_type=_f32)
    o_ref[...] = o2 + b3_ref[...]


_TC_PARAMS = pltpu.CompilerParams(dimension_semantics=("arbitrary",))


def _row_spec(bn, width):
    return pl.BlockSpec((bn, width), lambda i: (i, 0))


def _full_spec(shape):
    nd = len(shape)
    return pl.BlockSpec(shape, lambda i: (0,) * nd)


def _run_pre0(x, wenc, benc, lng, lnb, wg, as_a, ad_a):
    grid = (N // BN,)
    return pl.pallas_call(
        _pre0_body,
        grid=grid,
        in_specs=[_row_spec(BN, HID)] + [_full_spec(a.shape) for a in
                                         (wenc, benc, lng, lnb, wg, as_a, ad_a)],
        out_specs=[_row_spec(BN, HID), _row_spec(BN, HID),
                   _row_spec(BN, 16), _row_spec(BN, 16), _full_spec((1, 16))],
        out_shape=[jax.ShapeDtypeStruct((N, HID), _f32),
                   jax.ShapeDtypeStruct((N, HID), _f32),
                   jax.ShapeDtypeStruct((N, 16), _f32),
                   jax.ShapeDtypeStruct((N, 16), _f32),
                   jax.ShapeDtypeStruct((1, 16), _f32)],
        compiler_params=_TC_PARAMS,
    )(x, wenc, benc, lng, lnb, wg, as_a, ad_a)


def _run_pre_next(use_res, y, st, hp, gng, gnb, wres, bres, wg, as_a, ad_a):
    grid = (N // BN,)
    return pl.pallas_call(
        functools.partial(_pre_next_body, use_res),
        grid=grid,
        in_specs=[_row_spec(BN, HID), _full_spec((1, HID)), _row_spec(BN, HID)]
        + [_full_spec(a.shape) for a in (gng, gnb, wres, bres, wg, as_a, ad_a)],
        out_specs=[_row_spec(BN, HID), _row_spec(BN, HID),
                   _row_spec(BN, 16), _row_spec(BN, 16), _full_spec((1, 16))],
        out_shape=[jax.ShapeDtypeStruct((N, HID), _f32),
                   jax.ShapeDtypeStruct((N, HID), _f32),
                   jax.ShapeDtypeStruct((N, 16), _f32),
                   jax.ShapeDtypeStruct((N, 16), _f32),
                   jax.ShapeDtypeStruct((1, 16), _f32)],
        compiler_params=_TC_PARAMS,
    )(y, st, hp, gng, gnb, wres, bres, wg, as_a, ad_a)


def _run_post(y, st, hp, gng, gnb, wres, bres):
    grid = (N // BN,)
    return pl.pallas_call(
        _post_body,
        grid=grid,
        in_specs=[_row_spec(BN, HID), _full_spec((1, HID)), _row_spec(BN, HID)]
        + [_full_spec(a.shape) for a in (gng, gnb, wres, bres)],
        out_specs=_row_spec(BN, HID),
        out_shape=jax.ShapeDtypeStruct((N, HID), _f32),
        compiler_params=_TC_PARAMS,
    )(y, st, hp, gng, gnb, wres, bres)


def _run_comb(acc, den, xh, as16, ad16, m, bias, bmat):
    grid = (N // BN,)
    return pl.pallas_call(
        _comb_body,
        grid=grid,
        in_specs=[pl.BlockSpec((2, BN, HID), lambda i: (0, i, 0)),
                  pl.BlockSpec((2, BN, 16), lambda i: (0, i, 0)),
                  _row_spec(BN, HID), _row_spec(BN, 16), _row_spec(BN, 16),
                  _full_spec((1, 16)), _full_spec((1, HID)), _full_spec((8, HID))],
        out_specs=[_row_spec(BN, HID), _full_spec((1, HID))],
        out_shape=[jax.ShapeDtypeStruct((N, HID), _f32),
                   jax.ShapeDtypeStruct((1, HID), _f32)],
        compiler_params=_TC_PARAMS,
    )(acc, den, xh, as16, ad16, m, bias, bmat)


def _run_cls(sf, df, ea, weights, ebase, elen):
    grid = (elen // BE,)
    eoff = ebase // BE
    return pl.pallas_call(
        _cls_body,
        grid=grid,
        in_specs=[_row_spec(BE, HID), _row_spec(BE, HID),
                  pl.BlockSpec((BE, 16), lambda i: (i + eoff, 0))]
        + [_full_spec(w.shape) for w in weights],
        out_specs=pl.BlockSpec((2, BE), lambda i: (0, i)),
        out_shape=jax.ShapeDtypeStruct((2, elen), _f32),
        compiler_params=_TC_PARAMS,
    )(sf, df, ea, *weights)


def _sc_gat_body(n_heads, ei_h, as_h, ad_h, xh_h, m_h, z128_h, z16_h,
                 acc_o, den_o,
                 sidx, didx, asg, adg, wv, xg, mv, acc_sp, den_sp, gsem, ssem):
    c = lax.axis_index("c")
    s = lax.axis_index("s")
    r0 = s * _RPT

    @pl.when(s < 15)
    def _():
        pltpu.sync_copy(z128_h.at[pl.ds(0, _RPT)], acc_sp.at[pl.ds(r0, _RPT)])
        pltpu.sync_copy(z16_h.at[pl.ds(0, _RPT)], den_sp.at[pl.ds(r0, _RPT)])

    @pl.when(s == 15)
    def _():
        pltpu.sync_copy(z128_h.at[pl.ds(0, _RLAST)], acc_sp.at[pl.ds(r0, _RLAST)])
        pltpu.sync_copy(z16_h.at[pl.ds(0, _RLAST)], den_sp.at[pl.ds(r0, _RLAST)])

    pltpu.sync_copy(m_h, mv)
    plsc.subcore_barrier()
    mvec = mv[0]
    nch = jnp.where(s < 2, 79, 78)

    def issue(j, b):
        base = c * _EC + (16 * j + s) * _CHUNKG
        pltpu.sync_copy(ei_h.at[0, pl.ds(base, _CHUNKG)], sidx[b])
        pltpu.sync_copy(ei_h.at[1, pl.ds(base, _CHUNKG)], didx[b])
        pltpu.async_copy(as_h.at[sidx[b]], asg[b], gsem[b])
        pltpu.async_copy(ad_h.at[didx[b]], adg[b], gsem[b])
        pltpu.async_copy(xh_h.at[sidx[b]], xg[b], gsem[b])

    def wait_gathers(b):
        pltpu.make_async_copy(as_h.at[sidx[b]], asg[b], gsem[b]).wait()
        pltpu.make_async_copy(ad_h.at[didx[b]], adg[b], gsem[b]).wait()
        pltpu.make_async_copy(xh_h.at[sidx[b]], xg[b], gsem[b]).wait()

    def wait_scatters(b):
        pltpu.make_async_copy(wv[b], den_sp.at[didx[b]], ssem[b]).wait()
        pltpu.make_async_copy(xg[b], acc_sp.at[didx[b]], ssem[b]).wait()

    def compute_and_scatter(b):
        @plsc.parallel_loop(0, _CHUNKG, 1, unroll=4)
        def _(r):
            a = asg[b][r]
            d = adg[b][r]
            t = a + d
            u = d + mvec
            w = jnp.exp(jnp.maximum(t, 0.2 * t) - jnp.maximum(u, 0.2 * u))
            wv[b][r] = w
            for k in range(8):
                kk = k if n_heads == 8 else 0
                xg[b][r, pl.ds(16 * k, 16)] = xg[b][r, pl.ds(16 * k, 16)] * w[kk]

        pltpu.async_copy(wv[b], den_sp.at[didx[b]], ssem[b], add=True)
        pltpu.async_copy(xg[b], acc_sp.at[didx[b]], ssem[b], add=True)

    issue(0, 0)

    def pair(jj, carry):
        j0 = 2 * jj
        j1 = j0 + 1
        j2 = j0 + 2

        @pl.when(j1 < nch)
        def _():
            @pl.when(j1 >= 3)
            def _():
                wait_scatters(1)
            issue(j1, 1)

        @pl.when(j0 < nch)
        def _():
            wait_gathers(0)
            compute_and_scatter(0)

        @pl.when(j2 < nch)
        def _():
            wait_scatters(0)
            issue(j2, 0)

        @pl.when(j1 < nch)
        def _():
            wait_gathers(1)
            compute_and_scatter(1)

        return carry

    lax.fori_loop(0, 40, pair, 0)
    wait_scatters(0)
    wait_scatters(1)
    plsc.subcore_barrier()

    @pl.when(s < 15)
    def _():
        pltpu.sync_copy(acc_sp.at[pl.ds(r0, _RPT)], acc_o.at[c, pl.ds(r0, _RPT)])
        pltpu.sync_copy(den_sp.at[pl.ds(r0, _RPT)], den_o.at[c, pl.ds(r0, _RPT)])

    @pl.when(s == 15)
    def _():
        pltpu.sync_copy(acc_sp.at[pl.ds(r0, _RLAST)], acc_o.at[c, pl.ds(r0, _RLAST)])
        pltpu.sync_copy(den_sp.at[pl.ds(r0, _RLAST)], den_o.at[c, pl.ds(r0, _RLAST)])


def _run_sc_gat(n_heads, edge_index, as16, ad16, xh, m16, z128, z16):
    mesh = plsc.VectorSubcoreMesh(core_axis_name="c", subcore_axis_name="s")
    fn = functools.partial(
        pl.kernel,
        out_type=[jax.ShapeDtypeStruct((2, N, HID), _f32),
                  jax.ShapeDtypeStruct((2, N, 16), _f32)],
        mesh=mesh,
        scratch_types=[
            [pltpu.VMEM((_CHUNKG,), jnp.int32)] * 2,
            [pltpu.VMEM((_CHUNKG,), jnp.int32)] * 2,
            [pltpu.VMEM((_CHUNKG, 16), _f32)] * 2,
            [pltpu.VMEM((_CHUNKG, 16), _f32)] * 2,
            [pltpu.VMEM((_CHUNKG, 16), _f32)] * 2,
            [pltpu.VMEM((_CHUNKG, HID), _f32)] * 2,
            pltpu.VMEM((1, 16), _f32),
            pltpu.VMEM_SHARED((N, HID), _f32),
            pltpu.VMEM_SHARED((N, 16), _f32),
            [pltpu.SemaphoreType.DMA] * 2,
            [pltpu.SemaphoreType.DMA] * 2,
        ],
        compiler_params=pltpu.CompilerParams(use_tc_tiling_on_sc=False),
    )(functools.partial(_sc_gat_body, n_heads))
    return fn(edge_index, as16, ad16, xh, m16, z128, z16)


def _sc_gather_body(estart, ehalf, ei_h, h_h, sf_o, df_o, sidx, didx, sbuf, dbuf,
                    gsem, wsem):
    c = lax.axis_index("c")
    s = lax.axis_index("s")
    npertile = ehalf // 2 // _CHUNKG // 16
    nrem = ehalf // 2 // _CHUNKG - npertile * 16
    nch = jnp.where(s < nrem, npertile + 1, npertile)

    def base_of(j):
        return c * (ehalf // 2) + (16 * j + s) * _CHUNKG

    def issue(j, b):
        base = base_of(j)
        pltpu.sync_copy(ei_h.at[0, pl.ds(estart + base, _CHUNKG)], sidx[b])
        pltpu.sync_copy(ei_h.at[1, pl.ds(estart + base, _CHUNKG)], didx[b])
        pltpu.async_copy(h_h.at[sidx[b]], sbuf[b], gsem[b])
        pltpu.async_copy(h_h.at[didx[b]], dbuf[b], gsem[b])

    def wait_gathers(b):
        pltpu.make_async_copy(h_h.at[sidx[b]], sbuf[b], gsem[b]).wait()
        pltpu.make_async_copy(h_h.at[didx[b]], dbuf[b], gsem[b]).wait()

    def writeback(j, b):
        base = base_of(j)
        pltpu.async_copy(sbuf[b], sf_o.at[pl.ds(base, _CHUNKG)], wsem[b])
        pltpu.async_copy(dbuf[b], df_o.at[pl.ds(base, _CHUNKG)], wsem[b])

    def wait_writeback(j, b):
        base = base_of(j)
        pltpu.make_async_copy(sbuf[b], sf_o.at[pl.ds(base, _CHUNKG)], wsem[b]).wait()
        pltpu.make_async_copy(dbuf[b], df_o.at[pl.ds(base, _CHUNKG)], wsem[b]).wait()

    issue(0, 0)

    def pair(jj, carry):
        j0 = 2 * jj
        j1 = j0 + 1
        j2 = j0 + 2

        @pl.when(j1 < nch)
        def _():
            @pl.when(j1 >= 3)
            def _():
                wait_writeback(j1, 1)
            issue(j1, 1)

        wait_gathers(0)
        writeback(j0, 0)

        @pl.when(j2 < nch)
        def _():
            wait_writeback(j2, 0)
            issue(j2, 0)

        @pl.when(j1 < nch)
        def _():
            wait_gathers(1)
            writeback(j1, 1)

        return carry

    lax.fori_loop(0, (npertile + 2) // 2, pair, 0)
    wait_writeback(0, 0)
    wait_writeback(0, 1)


def _run_sc_gather(edge_index, h, estart, ehalf):
    mesh = plsc.VectorSubcoreMesh(core_axis_name="c", subcore_axis_name="s")
    fn = functools.partial(
        pl.kernel,
        out_type=[jax.ShapeDtypeStruct((ehalf, HID), _f32),
                  jax.ShapeDtypeStruct((ehalf, HID), _f32)],
        mesh=mesh,
        scratch_types=[
            [pltpu.VMEM((_CHUNKG,), jnp.int32)] * 2,
            [pltpu.VMEM((_CHUNKG,), jnp.int32)] * 2,
            [pltpu.VMEM((_CHUNKG, HID), _f32)] * 2,
            [pltpu.VMEM((_CHUNKG, HID), _f32)] * 2,
            [pltpu.SemaphoreType.DMA] * 2,
            [pltpu.SemaphoreType.DMA] * 2,
        ],
    )(functools.partial(_sc_gather_body, estart, ehalf))
    return fn(edge_index, h)


def _att_matrix(att):
    heads, ch = att.shape
    eye = jnp.eye(heads, dtype=_f32)
    r = jnp.repeat(eye, ch, axis=0)
    a = att.reshape(-1, 1) * r
    if heads < 8:
        a = jnp.pad(a, ((0, 0), (0, 8 - heads)))
    return a


def kernel(x, edge_index, edge_attr, params):
    p = params

    gat = p['gat']
    heads_cfg = [8, 8, 1]
    as_a = [_att_matrix(gat[i]['att_src']) for i in range(3)]
    ad_a = [_att_matrix(gat[i]['att_dst']) for i in range(3)]
    t8 = jnp.repeat(jnp.eye(8, dtype=_f32), 16, axis=1)
    t1 = jnp.zeros((8, HID), _f32).at[0, :].set(1.0)
    bmats = [t8, t8, t1]

    z128 = jnp.zeros((_RLAST, HID), _f32)
    z16 = jnp.zeros((_RLAST, 16), _f32)

    def row(v):
        return v.reshape(1, -1)

    ne = p['node_enc']
    h0, xh0, as0, ad0, m0 = _run_pre0(
        x, ne['lin']['w'], row(ne['lin']['b']), row(ne['ln_g']), row(ne['ln_b']),
        gat[0]['w'], as_a[0], ad_a[0])

    hs = [h0]
    xh, as16, ad16, m = xh0, as0, ad0, m0
    y = st = None
    for i in range(3):
        acc, den = _run_sc_gat(heads_cfg[i], edge_index, as16, ad16, xh, m,
                               z128, z16)
        y, st = _run_comb(acc, den, xh, as16, ad16, m,
                          row(gat[i]['bias']), bmats[i])
        nm = p['norms'][i]
        if i < 2:
            j = i + 1
            use_res = j > 1
            wres = p['res'][j - 2]['w'] if use_res else jnp.zeros((HID, HID), _f32)
            bres = row(p['res'][j - 2]['b']) if use_res else jnp.zeros((1, HID), _f32)
            hj, xh, as16, ad16, m = _run_pre_next(
                use_res, y, st, hs[-1], row(nm['g']), row(nm['b']),
                wres, bres, gat[j]['w'], as_a[j], ad_a[j])
            hs.append(hj)
        else:
            hf = _run_post(y, st, hs[-1], row(nm['g']), row(nm['b']),
                           p['res'][1]['w'], row(p['res'][1]['b']))

    c = p['cls']
    eg_w = p['edge_gate']['w']
    l1_w = c['lin1']['w']
    ee = p['edge_enc']
    weights = [
        ee['lin']['w'], row(ee['lin']['b']), row(ee['ln_g']), row(ee['ln_b']),
        eg_w[:HID], eg_w[HID:2 * HID], eg_w[2 * HID:], row(p['edge_gate']['b']),
        l1_w[:HID], l1_w[HID:], row(c['lin1']['b']), row(c['ln1_g']), row(c['ln1_b']),
        c['lin2']['w'], row(c['lin2']['b']), row(c['ln2_g']), row(c['ln2_b']),
        c['lin3']['w'].T, c['lin3']['b'].reshape(2, 1),
    ]
    eh = E // 2
    sf0, df0 = _run_sc_gather(edge_index, hf, 0, eh)
    sf1, df1 = _run_sc_gather(edge_index, hf, eh, eh)
    o0 = _run_cls(sf0, df0, edge_attr, weights, 0, eh)
    o1 = _run_cls(sf1, df1, edge_attr, weights, eh, eh)
    return jnp.concatenate([o0, o1], axis=1).T


# speedup vs baseline: 1.1459x; 1.1204x over previous
import functools

import jax
import jax.numpy as jnp
from jax import lax
from jax.experimental import pallas as pl
from jax.experimental.pallas import tpu as pltpu
from jax.experimental.pallas import tpu_sc as plsc

N = 10000
E = 160000
HID = 128
EPS = 1e-5

BN = 1000
BE = 1280

_EC = E // 2
_CHUNK = 128
_CHUNKG = 64
_NCHUNK = _EC // _CHUNK
_NCHUNKG = _EC // _CHUNKG
_RPT = 624
_RLAST = N - 15 * _RPT

_f32 = jnp.float32


def _ln_rows(t, g, b):
    mu = jnp.mean(t, axis=-1, keepdims=True)
    var = jnp.mean((t - mu) ** 2, axis=-1, keepdims=True)
    return (t - mu) / jnp.sqrt(var + EPS) * g + b


def _layer_prep(h, wg_ref, as_a_ref, ad_a_ref, xh_ref, as_ref, ad_ref, m_ref, pid):
    xh = jnp.dot(h, wg_ref[...], preferred_element_type=_f32)
    xh_ref[...] = xh
    a_s = jnp.dot(xh, as_a_ref[...], preferred_element_type=_f32)
    a_d = jnp.dot(xh, ad_a_ref[...], preferred_element_type=_f32)
    z = jnp.zeros_like(a_s)
    as_ref[...] = jnp.concatenate([a_s, z], axis=1)
    ad_ref[...] = jnp.concatenate([a_d, z], axis=1)
    bm = jnp.max(a_s, axis=0, keepdims=True)
    bm16 = jnp.concatenate([bm, jnp.zeros_like(bm)], axis=1)

    @pl.when(pid == 0)
    def _():
        m_ref[...] = bm16

    @pl.when(pid != 0)
    def _():
        m_ref[...] = jnp.maximum(m_ref[...], bm16)


def _pre0_body(x_ref, wenc_ref, benc_ref, lng_ref, lnb_ref,
               wg_ref, as_a_ref, ad_a_ref,
               h_ref, xh_ref, as_ref, ad_ref, m_ref):
    pid = pl.program_id(0)
    t = jnp.dot(x_ref[...], wenc_ref[...], preferred_element_type=_f32) + benc_ref[...]
    h = jnp.maximum(_ln_rows(t, lng_ref[...], lnb_ref[...]), 0.0)
    h_ref[...] = h
    _layer_prep(h, wg_ref, as_a_ref, ad_a_ref, xh_ref, as_ref, ad_ref, m_ref, pid)


def _pre_next_body(use_res, y_ref, st_ref, hp_ref, gng_ref, gnb_ref,
                   wres_ref, bres_ref, wg_ref, as_a_ref, ad_a_ref,
                   h_ref, xh_ref, as_ref, ad_ref, m_ref):
    pid = pl.program_id(0)
    s1 = st_ref[0:1, 0:1]
    s2 = st_ref[0:1, 1:2]
    mean = s1 / (N * HID)
    var = jnp.maximum(s2 / (N * HID) - mean * mean, 0.0)
    g = (y_ref[...] - mean) / (jnp.sqrt(var) + EPS) * gng_ref[...] + gnb_ref[...]
    if use_res:
        g = g + jnp.dot(hp_ref[...], wres_ref[...], preferred_element_type=_f32) + bres_ref[...]
    h = jnp.maximum(g, 0.0)
    h_ref[...] = h
    _layer_prep(h, wg_ref, as_a_ref, ad_a_ref, xh_ref, as_ref, ad_ref, m_ref, pid)


def _post_body(y_ref, st_ref, hp_ref, gng_ref, gnb_ref, wres_ref, bres_ref, h_ref):
    s1 = st_ref[0:1, 0:1]
    s2 = st_ref[0:1, 1:2]
    mean = s1 / (N * HID)
    var = jnp.maximum(s2 / (N * HID) - mean * mean, 0.0)
    g = (y_ref[...] - mean) / (jnp.sqrt(var) + EPS) * gng_ref[...] + gnb_ref[...]
    g = g + jnp.dot(hp_ref[...], wres_ref[...], preferred_element_type=_f32) + bres_ref[...]
    h_ref[...] = jnp.maximum(g, 0.0)


def _comb_body(acc_ref, den_ref, xh_ref, as_ref, ad_ref, m_ref, bias_ref, t_ref,
               y_ref, st_ref):
    pid = pl.program_id(0)
    acc = acc_ref[0] + acc_ref[1]
    den = den_ref[0, :, 0:8] + den_ref[1, :, 0:8]
    a_s = as_ref[:, 0:8]
    a_d = ad_ref[:, 0:8]
    t = a_s + a_d
    lr = jnp.maximum(t, 0.2 * t)
    u = a_d + m_ref[0:1, 0:8]
    shift = jnp.maximum(u, 0.2 * u)
    wself = jnp.exp(lr - shift)
    dinv = 1.0 / (den + wself + 1e-16)
    bmat = t_ref[...]
    y = (acc + xh_ref[...] * jnp.dot(wself, bmat, preferred_element_type=_f32)) \
        * jnp.dot(dinv, bmat, preferred_element_type=_f32) + bias_ref[...]
    y_ref[...] = y
    s1 = jnp.sum(y)
    s2 = jnp.sum(y * y)
    lanes = lax.broadcasted_iota(jnp.int32, (1, HID), 1)
    vec = jnp.where(lanes == 0, s1, jnp.where(lanes == 1, s2, 0.0))

    @pl.when(pid == 0)
    def _():
        st_ref[...] = vec

    @pl.when(pid != 0)
    def _():
        st_ref[...] = st_ref[...] + vec


def _cls_body(sf_ref, df_ref, ea_ref,
              we_ref, be_ref, elng_ref, elnb_ref,
              wgs_ref, wgd_ref, wge_ref, bg_ref,
              w1s_ref, w1d_ref, b1_ref, l1g_ref, l1b_ref,
              w2_ref, b2_ref, l2g_ref, l2b_ref,
              w3_ref, b3_ref,
              o_ref):
    sf = sf_ref[...]
    df = df_ref[...]
    ef = jnp.dot(ea_ref[...], we_ref[...], preferred_element_type=_f32) + be_ref[...]
    ef = jnp.maximum(_ln_rows(ef, elng_ref[...], elnb_ref[...]), 0.0)
    gate = jnp.dot(sf, wgs_ref[...], preferred_element_type=_f32) \
        + jnp.dot(df, wgd_ref[...], preferred_element_type=_f32) \
        + jnp.dot(ef, wge_ref[...], preferred_element_type=_f32) + bg_ref[...]
    gate = jax.nn.sigmoid(gate)
    gef = gate * ef
    s2 = sf + gef
    d2 = df + gef
    z1 = jnp.dot(s2, w1s_ref[...], preferred_element_type=_f32) \
        + jnp.dot(d2, w1d_ref[...], preferred_element_type=_f32) + b1_ref[...]
    z1 = jnp.maximum(_ln_rows(z1, l1g_ref[...], l1b_ref[...]), 0.0)
    z2 = jnp.dot(z1, w2_ref[...], preferred_element_type=_f32) + b2_ref[...]
    z2 = jnp.maximum(_ln_rows(z2, l2g_ref[...], l2b_ref[...]), 0.0)
    o2 = lax.dot_general(w3_ref[...], z2, (((1,), (1,)), ((), ())),
                         preferred_element_type=_f32)
    o_ref[...] = o2 + b3_ref[...]


_TC_PARAMS = pltpu.CompilerParams(dimension_semantics=("arbitrary",))


def _row_spec(bn, width):
    return pl.BlockSpec((bn, width), lambda i: (i, 0))


def _full_spec(shape):
    nd = len(shape)
    return pl.BlockSpec(shape, lambda i: (0,) * nd)


def _run_pre0(x, wenc, benc, lng, lnb, wg, as_a, ad_a):
    grid = (N // BN,)
    return pl.pallas_call(
        _pre0_body,
        grid=grid,
        in_specs=[_row_spec(BN, HID)] + [_full_spec(a.shape) for a in
                                         (wenc, benc, lng, lnb, wg, as_a, ad_a)],
        out_specs=[_row_spec(BN, HID), _row_spec(BN, HID),
                   _row_spec(BN, 16), _row_spec(BN, 16), _full_spec((1, 16))],
        out_shape=[jax.ShapeDtypeStruct((N, HID), _f32),
                   jax.ShapeDtypeStruct((N, HID), _f32),
                   jax.ShapeDtypeStruct((N, 16), _f32),
                   jax.ShapeDtypeStruct((N, 16), _f32),
                   jax.ShapeDtypeStruct((1, 16), _f32)],
        compiler_params=_TC_PARAMS,
    )(x, wenc, benc, lng, lnb, wg, as_a, ad_a)


def _run_pre_next(use_res, y, st, hp, gng, gnb, wres, bres, wg, as_a, ad_a):
    grid = (N // BN,)
    return pl.pallas_call(
        functools.partial(_pre_next_body, use_res),
        grid=grid,
        in_specs=[_row_spec(BN, HID), _full_spec((1, HID)), _row_spec(BN, HID)]
        + [_full_spec(a.shape) for a in (gng, gnb, wres, bres, wg, as_a, ad_a)],
        out_specs=[_row_spec(BN, HID), _row_spec(BN, HID),
                   _row_spec(BN, 16), _row_spec(BN, 16), _full_spec((1, 16))],
        out_shape=[jax.ShapeDtypeStruct((N, HID), _f32),
                   jax.ShapeDtypeStruct((N, HID), _f32),
                   jax.ShapeDtypeStruct((N, 16), _f32),
                   jax.ShapeDtypeStruct((N, 16), _f32),
                   jax.ShapeDtypeStruct((1, 16), _f32)],
        compiler_params=_TC_PARAMS,
    )(y, st, hp, gng, gnb, wres, bres, wg, as_a, ad_a)


def _run_post(y, st, hp, gng, gnb, wres, bres):
    grid = (N // BN,)
    return pl.pallas_call(
        _post_body,
        grid=grid,
        in_specs=[_row_spec(BN, HID), _full_spec((1, HID)), _row_spec(BN, HID)]
        + [_full_spec(a.shape) for a in (gng, gnb, wres, bres)],
        out_specs=_row_spec(BN, HID),
        out_shape=jax.ShapeDtypeStruct((N, HID), _f32),
        compiler_params=_TC_PARAMS,
    )(y, st, hp, gng, gnb, wres, bres)


def _run_comb(acc, den, xh, as16, ad16, m, bias, bmat):
    grid = (N // BN,)
    return pl.pallas_call(
        _comb_body,
        grid=grid,
        in_specs=[pl.BlockSpec((2, BN, HID), lambda i: (0, i, 0)),
                  pl.BlockSpec((2, BN, 16), lambda i: (0, i, 0)),
                  _row_spec(BN, HID), _row_spec(BN, 16), _row_spec(BN, 16),
                  _full_spec((1, 16)), _full_spec((1, HID)), _full_spec((8, HID))],
        out_specs=[_row_spec(BN, HID), _full_spec((1, HID))],
        out_shape=[jax.ShapeDtypeStruct((N, HID), _f32),
                   jax.ShapeDtypeStruct((1, HID), _f32)],
        compiler_params=_TC_PARAMS,
    )(acc, den, xh, as16, ad16, m, bias, bmat)


def _run_cls(sf, df, ea, weights, ebase, elen):
    grid = (elen // BE,)
    eoff = ebase // BE
    return pl.pallas_call(
        _cls_body,
        grid=grid,
        in_specs=[_row_spec(BE, HID), _row_spec(BE, HID),
                  pl.BlockSpec((BE, 16), lambda i: (i + eoff, 0))]
        + [_full_spec(w.shape) for w in weights],
        out_specs=pl.BlockSpec((2, BE), lambda i: (0, i)),
        out_shape=jax.ShapeDtypeStruct((2, elen), _f32),
        compiler_params=_TC_PARAMS,
    )(sf, df, ea, *weights)


def _sc_gat_body(n_heads, ei_h, as_h, ad_h, xh_h, m_h, z128_h, z16_h,
                 acc_o, den_o,
                 sidx, didx, asg, adg, wv, xg, mv, acc_sp, den_sp, gsem, ssem):
    c = lax.axis_index("c")
    s = lax.axis_index("s")
    r0 = s * _RPT

    @pl.when(s < 15)
    def _():
        pltpu.sync_copy(z128_h.at[pl.ds(0, _RPT)], acc_sp.at[pl.ds(r0, _RPT)])
        pltpu.sync_copy(z16_h.at[pl.ds(0, _RPT)], den_sp.at[pl.ds(r0, _RPT)])

    @pl.when(s == 15)
    def _():
        pltpu.sync_copy(z128_h.at[pl.ds(0, _RLAST)], acc_sp.at[pl.ds(r0, _RLAST)])
        pltpu.sync_copy(z16_h.at[pl.ds(0, _RLAST)], den_sp.at[pl.ds(r0, _RLAST)])

    pltpu.sync_copy(m_h, mv)
    plsc.subcore_barrier()
    mvec = mv[0]
    nch = jnp.where(s < 2, 79, 78)

    def issue(j, b):
        base = c * _EC + (16 * j + s) * _CHUNKG
        pltpu.sync_copy(ei_h.at[0, pl.ds(base, _CHUNKG)], sidx[b])
        pltpu.sync_copy(ei_h.at[1, pl.ds(base, _CHUNKG)], didx[b])
        pltpu.async_copy(as_h.at[sidx[b]], asg[b], gsem[b])
        pltpu.async_copy(ad_h.at[didx[b]], adg[b], gsem[b])
        pltpu.async_copy(xh_h.at[sidx[b]], xg[b], gsem[b])

    def wait_gathers(b):
        pltpu.make_async_copy(as_h.at[sidx[b]], asg[b], gsem[b]).wait()
        pltpu.make_async_copy(ad_h.at[didx[b]], adg[b], gsem[b]).wait()
        pltpu.make_async_copy(xh_h.at[sidx[b]], xg[b], gsem[b]).wait()

    def wait_scatters(b):
        pltpu.make_async_copy(wv[b], den_sp.at[didx[b]], ssem[b]).wait()
        pltpu.make_async_copy(xg[b], acc_sp.at[didx[b]], ssem[b]).wait()

    def compute_and_scatter(b):
        @plsc.parallel_loop(0, _CHUNKG, 1, unroll=4)
        def _(r):
            a = asg[b][r]
            d = adg[b][r]
            t = a + d
            u = d + mvec
            w = jnp.exp(jnp.maximum(t, 0.2 * t) - jnp.maximum(u, 0.2 * u))
            wv[b][r] = w
            for k in range(8):
                kk = k if n_heads == 8 else 0
                xg[b][r, pl.ds(16 * k, 16)] = xg[b][r, pl.ds(16 * k, 16)] * w[kk]

        pltpu.async_copy(wv[b], den_sp.at[didx[b]], ssem[b], add=True)
        pltpu.async_copy(xg[b], acc_sp.at[didx[b]], ssem[b], add=True)

    issue(0, 0)

    def pair(jj, carry):
        j0 = 2 * jj
        j1 = j0 + 1
        j2 = j0 + 2

        @pl.when(j1 < nch)
        def _():
            @pl.when(j1 >= 3)
            def _():
                wait_scatters(1)
            issue(j1, 1)

        @pl.when(j0 < nch)
        def _():
            wait_gathers(0)
            compute_and_scatter(0)

        @pl.when(j2 < nch)
        def _():
            wait_scatters(0)
            issue(j2, 0)

        @pl.when(j1 < nch)
        def _():
            wait_gathers(1)
            compute_and_scatter(1)

        return carry

    lax.fori_loop(0, 40, pair, 0)
    wait_scatters(0)
    wait_scatters(1)
    plsc.subcore_barrier()

    @pl.when(s < 15)
    def _():
        pltpu.sync_copy(acc_sp.at[pl.ds(r0, _RPT)], acc_o.at[c, pl.ds(r0, _RPT)])
        pltpu.sync_copy(den_sp.at[pl.ds(r0, _RPT)], den_o.at[c, pl.ds(r0, _RPT)])

    @pl.when(s == 15)
    def _():
        pltpu.sync_copy(acc_sp.at[pl.ds(r0, _RLAST)], acc_o.at[c, pl.ds(r0, _RLAST)])
        pltpu.sync_copy(den_sp.at[pl.ds(r0, _RLAST)], den_o.at[c, pl.ds(r0, _RLAST)])


def _run_sc_gat(n_heads, edge_index, as16, ad16, xh, m16, z128, z16):
    mesh = plsc.VectorSubcoreMesh(core_axis_name="c", subcore_axis_name="s")
    fn = functools.partial(
        pl.kernel,
        out_type=[jax.ShapeDtypeStruct((2, N, HID), _f32),
                  jax.ShapeDtypeStruct((2, N, 16), _f32)],
        mesh=mesh,
        scratch_types=[
            [pltpu.VMEM((_CHUNKG,), jnp.int32)] * 2,
            [pltpu.VMEM((_CHUNKG,), jnp.int32)] * 2,
            [pltpu.VMEM((_CHUNKG, 16), _f32)] * 2,
            [pltpu.VMEM((_CHUNKG, 16), _f32)] * 2,
            [pltpu.VMEM((_CHUNKG, 16), _f32)] * 2,
            [pltpu.VMEM((_CHUNKG, HID), _f32)] * 2,
            pltpu.VMEM((1, 16), _f32),
            pltpu.VMEM_SHARED((N, HID), _f32),
            pltpu.VMEM_SHARED((N, 16), _f32),
            [pltpu.SemaphoreType.DMA] * 2,
            [pltpu.SemaphoreType.DMA] * 2,
        ],
        compiler_params=pltpu.CompilerParams(use_tc_tiling_on_sc=False),
    )(functools.partial(_sc_gat_body, n_heads))
    return fn(edge_index, as16, ad16, xh, m16, z128, z16)


def _sc_gather_body(estart, ehalf, ei_h, h_h, sf_o, df_o, sidx, didx, sbuf, dbuf,
                    gsem, wsem):
    c = lax.axis_index("c")
    s = lax.axis_index("s")
    npertile = ehalf // 2 // _CHUNKG // 16
    nrem = ehalf // 2 // _CHUNKG - npertile * 16
    nch = jnp.where(s < nrem, npertile + 1, npertile)

    def base_of(j):
        return c * (ehalf // 2) + (16 * j + s) * _CHUNKG

    def issue(j, b):
        base = base_of(j)
        pltpu.sync_copy(ei_h.at[0, pl.ds(estart + base, _CHUNKG)], sidx[b])
        pltpu.sync_copy(ei_h.at[1, pl.ds(estart + base, _CHUNKG)], didx[b])
        pltpu.async_copy(h_h.at[sidx[b]], sbuf[b], gsem[b])
        pltpu.async_copy(h_h.at[didx[b]], dbuf[b], gsem[b])

    def wait_gathers(b):
        pltpu.make_async_copy(h_h.at[sidx[b]], sbuf[b], gsem[b]).wait()
        pltpu.make_async_copy(h_h.at[didx[b]], dbuf[b], gsem[b]).wait()

    def writeback(j, b):
        base = base_of(j)
        pltpu.async_copy(sbuf[b], sf_o.at[pl.ds(base, _CHUNKG)], wsem[b])
        pltpu.async_copy(dbuf[b], df_o.at[pl.ds(base, _CHUNKG)], wsem[b])

    def wait_writeback(j, b):
        base = base_of(j)
        pltpu.make_async_copy(sbuf[b], sf_o.at[pl.ds(base, _CHUNKG)], wsem[b]).wait()
        pltpu.make_async_copy(dbuf[b], df_o.at[pl.ds(base, _CHUNKG)], wsem[b]).wait()

    issue(0, 0)

    def pair(jj, carry):
        j0 = 2 * jj
        j1 = j0 + 1
        j2 = j0 + 2

        @pl.when(j1 < nch)
        def _():
            @pl.when(j1 >= 3)
            def _():
                wait_writeback(j1, 1)
            issue(j1, 1)

        @pl.when(j0 < nch)
        def _():
            wait_gathers(0)
            writeback(j0, 0)

        @pl.when(j2 < nch)
        def _():
            wait_writeback(j2, 0)
            issue(j2, 0)

        @pl.when(j1 < nch)
        def _():
            wait_gathers(1)
            writeback(j1, 1)

        return carry

    max_nch = npertile + (1 if nrem else 0)
    lax.fori_loop(0, (max_nch + 1) // 2, pair, 0)
    wait_writeback(0, 0)
    wait_writeback(0, 1)


def _run_sc_gather(edge_index, h, estart, ehalf):
    mesh = plsc.VectorSubcoreMesh(core_axis_name="c", subcore_axis_name="s")
    fn = functools.partial(
        pl.kernel,
        out_type=[jax.ShapeDtypeStruct((ehalf, HID), _f32),
                  jax.ShapeDtypeStruct((ehalf, HID), _f32)],
        mesh=mesh,
        scratch_types=[
            [pltpu.VMEM((_CHUNKG,), jnp.int32)] * 2,
            [pltpu.VMEM((_CHUNKG,), jnp.int32)] * 2,
            [pltpu.VMEM((_CHUNKG, HID), _f32)] * 2,
            [pltpu.VMEM((_CHUNKG, HID), _f32)] * 2,
            [pltpu.SemaphoreType.DMA] * 2,
            [pltpu.SemaphoreType.DMA] * 2,
        ],
    )(functools.partial(_sc_gather_body, estart, ehalf))
    return fn(edge_index, h)


def _att_matrix(att):
    heads, ch = att.shape
    eye = jnp.eye(heads, dtype=_f32)
    r = jnp.repeat(eye, ch, axis=0)
    a = att.reshape(-1, 1) * r
    if heads < 8:
        a = jnp.pad(a, ((0, 0), (0, 8 - heads)))
    return a


def kernel(x, edge_index, edge_attr, params):
    p = params

    gat = p['gat']
    heads_cfg = [8, 8, 1]
    as_a = [_att_matrix(gat[i]['att_src']) for i in range(3)]
    ad_a = [_att_matrix(gat[i]['att_dst']) for i in range(3)]
    t8 = jnp.repeat(jnp.eye(8, dtype=_f32), 16, axis=1)
    t1 = jnp.zeros((8, HID), _f32).at[0, :].set(1.0)
    bmats = [t8, t8, t1]

    z128 = jnp.zeros((_RLAST, HID), _f32)
    z16 = jnp.zeros((_RLAST, 16), _f32)

    def row(v):
        return v.reshape(1, -1)

    ne = p['node_enc']
    h0, xh0, as0, ad0, m0 = _run_pre0(
        x, ne['lin']['w'], row(ne['lin']['b']), row(ne['ln_g']), row(ne['ln_b']),
        gat[0]['w'], as_a[0], ad_a[0])

    hs = [h0]
    xh, as16, ad16, m = xh0, as0, ad0, m0
    y = st = None
    for i in range(3):
        acc, den = _run_sc_gat(heads_cfg[i], edge_index, as16, ad16, xh, m,
                               z128, z16)
        y, st = _run_comb(acc, den, xh, as16, ad16, m,
                          row(gat[i]['bias']), bmats[i])
        nm = p['norms'][i]
        if i < 2:
            j = i + 1
            use_res = j > 1
            wres = p['res'][j - 2]['w'] if use_res else jnp.zeros((HID, HID), _f32)
            bres = row(p['res'][j - 2]['b']) if use_res else jnp.zeros((1, HID), _f32)
            hj, xh, as16, ad16, m = _run_pre_next(
                use_res, y, st, hs[-1], row(nm['g']), row(nm['b']),
                wres, bres, gat[j]['w'], as_a[j], ad_a[j])
            hs.append(hj)
        else:
            hf = _run_post(y, st, hs[-1], row(nm['g']), row(nm['b']),
                           p['res'][1]['w'], row(p['res'][1]['b']))

    c = p['cls']
    eg_w = p['edge_gate']['w']
    l1_w = c['lin1']['w']
    ee = p['edge_enc']
    weights = [
        ee['lin']['w'], row(ee['lin']['b']), row(ee['ln_g']), row(ee['ln_b']),
        eg_w[:HID], eg_w[HID:2 * HID], eg_w[2 * HID:], row(p['edge_gate']['b']),
        l1_w[:HID], l1_w[HID:], row(c['lin1']['b']), row(c['ln1_g']), row(c['ln1_b']),
        c['lin2']['w'], row(c['lin2']['b']), row(c['ln2_g']), row(c['ln2_b']),
        c['lin3']['w'].T, c['lin3']['b'].reshape(2, 1),
    ]
    eh0 = 64 * BE
    eh1 = E - eh0
    sf0, df0 = _run_sc_gather(edge_index, hf, 0, eh0)
    sf1, df1 = _run_sc_gather(edge_index, hf, eh0, eh1)
    o0 = _run_cls(sf0, df0, edge_attr, weights, 0, eh0)
    o1 = _run_cls(sf1, df1, edge_attr, weights, eh0, eh1)
    return jnp.concatenate([o0, o1], axis=1).T
